# speedup vs baseline: 1.2761x; 1.2761x over previous
import jax
import jax.numpy as jnp
from jax.experimental import pallas as pl
from jax.experimental.pallas import tpu as pltpu


def _fused_body(x_ref, w_ref, o_ref):
    B, P, D = x_ref.shape
    C = w_ref.shape[0]
    x = x_ref[...].reshape(B * P, D)
    w = w_ref[...]
    h = jax.lax.dot_general(
        x, w, (((1,), (1,)), ((), ())), preferred_element_type=jnp.float32
    ).reshape(B, P, C)
    m = jnp.max(h, axis=1, keepdims=True)
    iota_p = jax.lax.broadcasted_iota(jnp.int32, (B, P, C), 1)
    first = jnp.min(jnp.where(h == m, iota_p, P), axis=1)
    idx = jnp.where(m[:, 0, :] > 0, first, 0)
    onehot = (iota_p == idx[:, None, :]).astype(jnp.float32)
    out = jax.lax.dot_general(
        onehot.reshape(B * P, C), w, (((1,), (0,)), ((), ())),
        preferred_element_type=jnp.float32,
    )
    o_ref[...] = out.reshape(B, P, D)


def kernel(x, W):
    bsz, P, d = x.shape
    C = W.shape[0]
    B_BLK = 8
    grid = (bsz // B_BLK,)
    return pl.pallas_call(
        _fused_body,
        grid=grid,
        in_specs=[
            pl.BlockSpec((B_BLK, P, d), lambda i: (i, 0, 0)),
            pl.BlockSpec((C, d), lambda i: (0, 0)),
        ],
        out_specs=pl.BlockSpec((B_BLK, P, d), lambda i: (i, 0, 0)),
        out_shape=jax.ShapeDtypeStruct((bsz, P, d), x.dtype),
        compiler_params=pltpu.CompilerParams(
            dimension_semantics=("arbitrary",),
        ),
    )(x, W)
